# Initial kernel scaffold; baseline (speedup 1.0000x reference)
import jax
import jax.numpy as jnp
from jax.experimental import pallas as pl


def kernel(user_emb, item_emb, edge_index, users, items, neg_items):
    raise NotImplementedError("write your pallas kernel here")


# speedup vs baseline: 6.7782x; 6.7782x over previous
import functools

import jax
import jax.numpy as jnp
from jax import lax
from jax.experimental import pallas as pl
from jax.experimental.pallas import tpu as pltpu
from jax.experimental.pallas import tpu_sc as plsc

N_U = 30000
N_I = 20000
NN = 50000
EE = 800000
DD = 64
BB = 4096

H = 25000
CH = 1568
HP = 16 * CH
K = 128
NCHUNK = 391
PER_SUB = NCHUNK * K
EP = 16 * PER_SUB

_mesh = plsc.VectorSubcoreMesh(
    core_axis_name="c", subcore_axis_name="s", num_cores=2, num_subcores=16)


def _f32(shape):
    return jax.ShapeDtypeStruct(shape, jnp.float32)


@functools.partial(
    pl.kernel,
    out_type=_f32((NN, 16)),
    mesh=_mesh,
    compiler_params=pltpu.CompilerParams(use_tc_tiling_on_sc=False),
    scratch_types=[
        pltpu.VMEM_SHARED((HP, 16), jnp.float32),
        pltpu.VMEM((K,), jnp.int32),
        pltpu.VMEM((K,), jnp.int32),
        pltpu.VMEM((K, 16), jnp.float32),
    ],
)
def _deg_kernel(rowp, zrs16, deg16, dacc, rowv, idxv, onesv):
    c = lax.axis_index("c")
    s = lax.axis_index("s")
    c_lo = c * H
    pltpu.sync_copy(zrs16.at[pl.ds(s * CH, CH)], dacc.at[pl.ds(s * CH, CH)])
    pat = jnp.where(lax.iota(jnp.int32, 16) == 0,
                    jnp.float32(1.0), jnp.float32(0.0))
    for k in range(K):
        onesv[k, :] = pat
    plsc.subcore_barrier()

    def chunk(g, carry):
        base = s * PER_SUB + g * K
        pltpu.sync_copy(rowp.at[pl.ds(base, K)], rowv)
        for j in range(K // 16):
            r = rowv[pl.ds(j * 16, 16)]
            lo = r - c_lo
            ok = (lo >= 0) & (lo < H)
            idxv[pl.ds(j * 16, 16)] = jnp.where(ok, lo, H)
        pltpu.sync_copy(onesv, dacc.at[idxv], add=True)
        return carry

    lax.fori_loop(0, NCHUNK, chunk, 0)
    plsc.subcore_barrier()
    ob = jnp.minimum(s * CH, H - CH)
    pltpu.sync_copy(dacc.at[pl.ds(ob, CH)], deg16.at[pl.ds(c_lo + ob, CH)])


@functools.partial(
    pl.kernel,
    out_type=_f32((NN, DD)),
    mesh=_mesh,
    compiler_params=pltpu.CompilerParams(use_tc_tiling_on_sc=False),
    scratch_types=[
        pltpu.VMEM_SHARED((HP, DD), jnp.float32),
        pltpu.VMEM((K,), jnp.int32),
        pltpu.VMEM((K,), jnp.int32),
        pltpu.VMEM((K,), jnp.int32),
        pltpu.VMEM((K, DD), jnp.float32),
        pltpu.SemaphoreType.DMA,
    ],
)
def _spmm_kernel(y, colp, rowp, zrs, out, acc, colv, rowv, idxv, rowsbuf, sem):
    c = lax.axis_index("c")
    s = lax.axis_index("s")
    c_lo = c * H
    pltpu.sync_copy(zrs.at[pl.ds(s * CH, CH)], acc.at[pl.ds(s * CH, CH)])
    plsc.subcore_barrier()

    def chunk(g, carry):
        base = s * PER_SUB + g * K
        pltpu.sync_copy(colp.at[pl.ds(base, K)], colv)
        pltpu.sync_copy(rowp.at[pl.ds(base, K)], rowv)
        for j in range(K // 16):
            r = rowv[pl.ds(j * 16, 16)]
            lo = r - c_lo
            ok = (lo >= 0) & (lo < H)
            idxv[pl.ds(j * 16, 16)] = jnp.where(ok, lo, H)
        pltpu.async_copy(y.at[colv], rowsbuf, sem).wait()
        pltpu.sync_copy(rowsbuf, acc.at[idxv], add=True)
        return carry

    lax.fori_loop(0, NCHUNK, chunk, 0)
    plsc.subcore_barrier()
    ob = jnp.minimum(s * CH, H - CH)
    pltpu.sync_copy(acc.at[pl.ds(ob, CH)], out.at[pl.ds(c_lo + ob, CH)])


@functools.partial(
    pl.kernel,
    out_type=(_f32((BB, DD)), _f32((BB, DD)), _f32((BB, DD))),
    mesh=_mesh,
    compiler_params=pltpu.CompilerParams(use_tc_tiling_on_sc=False),
    scratch_types=[
        pltpu.VMEM((K,), jnp.int32),
        pltpu.VMEM((K, DD), jnp.float32),
        pltpu.SemaphoreType.DMA,
    ],
)
def _bpr_gather_kernel(S, users, items, negs, anc, pos, neg, idxv, buf, sem):
    c = lax.axis_index("c")
    s = lax.axis_index("s")
    base = (s * 2 + c) * K

    pltpu.sync_copy(users.at[pl.ds(base, K)], idxv)
    pltpu.async_copy(S.at[idxv], buf, sem).wait()
    pltpu.sync_copy(buf, anc.at[pl.ds(base, K)])

    for src, dst in ((items, pos), (negs, neg)):
        pltpu.sync_copy(src.at[pl.ds(base, K)], idxv)
        for j in range(K // 16):
            idxv[pl.ds(j * 16, 16)] = idxv[pl.ds(j * 16, 16)] + N_U
        pltpu.async_copy(S.at[idxv], buf, sem).wait()
        pltpu.sync_copy(buf, dst.at[pl.ds(base, K)])


_RB = 2000


def _prep_body(deg_ref, x0_ref, dis_ref, y0_ref):
    d = jnp.clip(lax.rsqrt(deg_ref[:, 0:1] + 1e-6), 0.0, 10.0)
    dis_ref[...] = d
    y0_ref[...] = d * x0_ref[...]


_prep = pl.pallas_call(
    _prep_body,
    grid=(NN // _RB,),
    in_specs=[
        pl.BlockSpec((_RB, 16), lambda i: (i, 0)),
        pl.BlockSpec((_RB, DD), lambda i: (i, 0)),
    ],
    out_specs=[
        pl.BlockSpec((_RB, 1), lambda i: (i, 0)),
        pl.BlockSpec((_RB, DD), lambda i: (i, 0)),
    ],
    out_shape=[_f32((NN, 1)), _f32((NN, DD))],
)


def _scale_body(acc_ref, dis_ref, s_ref, snew_ref, y_ref):
    d = dis_ref[...]
    da = d * acc_ref[...]
    snew_ref[...] = s_ref[...] + da
    y_ref[...] = d * da


_scale = pl.pallas_call(
    _scale_body,
    grid=(NN // _RB,),
    in_specs=[
        pl.BlockSpec((_RB, DD), lambda i: (i, 0)),
        pl.BlockSpec((_RB, 1), lambda i: (i, 0)),
        pl.BlockSpec((_RB, DD), lambda i: (i, 0)),
    ],
    out_specs=[
        pl.BlockSpec((_RB, DD), lambda i: (i, 0)),
        pl.BlockSpec((_RB, DD), lambda i: (i, 0)),
    ],
    out_shape=[_f32((NN, DD)), _f32((NN, DD))],
)


def _loss_body(a_ref, p_ref, n_ref, o_ref):
    a = a_ref[...]
    p = p_ref[...]
    n = n_ref[...]
    diff = jnp.sum(a * p, axis=-1) - jnp.sum(a * n, axis=-1)
    bpr = -jnp.sum(jnp.log(jax.nn.sigmoid(diff) + 1e-12)) / float(BB)
    reg = 0.5 * (jnp.sum(a * a) + jnp.sum(p * p) + jnp.sum(n * n)) / float(BB)
    o_ref[...] = (bpr + reg).reshape(1, 1)


_loss = pl.pallas_call(
    _loss_body,
    out_shape=_f32((1, 1)),
)


def kernel(user_emb, item_emb, edge_index, users, items, neg_items):
    x0 = jnp.concatenate([user_emb, item_emb], axis=0)
    pad = EP - EE
    colp = jnp.concatenate([edge_index[1], jnp.zeros((pad,), jnp.int32)])
    rowp = jnp.concatenate([edge_index[0], jnp.full((pad,), -1, jnp.int32)])
    zrs = jnp.zeros((HP, DD), jnp.float32)
    zrs16 = jnp.zeros((HP, 16), jnp.float32)

    deg16 = _deg_kernel(rowp, zrs16)
    dis, y = _prep(deg16, x0)
    s_sum = x0
    for _ in range(3):
        acc = _spmm_kernel(y, colp, rowp, zrs)
        s_sum, y = _scale(acc, dis, s_sum)
    anc, pos, neg = _bpr_gather_kernel(s_sum, users, items, neg_items)
    return _loss(anc, pos, neg)[0, 0]


# speedup vs baseline: 8.4090x; 1.2406x over previous
import functools

import jax
import jax.numpy as jnp
from jax import lax
from jax.experimental import pallas as pl
from jax.experimental.pallas import tpu as pltpu
from jax.experimental.pallas import tpu_sc as plsc

N_U = 30000
N_I = 20000
NN = 50000
EE = 800000
DD = 64
BB = 4096

H = 25000
CH = 1568
HP = 16 * CH
K = 64
NB = 6
BLK = NB * K
NBLK = 131
PER_SUB = NBLK * BLK
EP = 16 * PER_SUB

_mesh = plsc.VectorSubcoreMesh(
    core_axis_name="c", subcore_axis_name="s", num_cores=2, num_subcores=16)


def _f32(shape):
    return jax.ShapeDtypeStruct(shape, jnp.float32)


@functools.partial(
    pl.kernel,
    out_type=_f32((NN, 16)),
    mesh=_mesh,
    compiler_params=pltpu.CompilerParams(use_tc_tiling_on_sc=False),
    scratch_types=[
        pltpu.VMEM_SHARED((HP, 16), jnp.float32),
        pltpu.VMEM((2, BLK), jnp.int32),
        pltpu.VMEM((NB, K), jnp.int32),
        pltpu.VMEM((K, 16), jnp.float32),
        pltpu.SemaphoreType.DMA((2,)),
        pltpu.SemaphoreType.DMA((NB,)),
    ],
)
def _deg_kernel(rowp, zrs16, deg16, dacc, rowb, idxb, onesv, isem, ssem):
    c = lax.axis_index("c")
    s = lax.axis_index("s")
    c_lo = c * H
    ebase = s * PER_SUB
    pltpu.sync_copy(zrs16.at[pl.ds(s * CH, CH)], dacc.at[pl.ds(s * CH, CH)])
    pat = jnp.where(lax.iota(jnp.int32, 16) == 0,
                    jnp.float32(1.0), jnp.float32(0.0))
    for k in range(K):
        onesv[k, :] = pat
    plsc.subcore_barrier()

    pltpu.async_copy(rowp.at[pl.ds(ebase, BLK)], rowb.at[0], isem.at[0])

    def block(b, carry):
        p = jnp.bitwise_and(b, 1)
        pltpu.make_async_copy(
            rowp.at[pl.ds(ebase + b * BLK, BLK)], rowb.at[p], isem.at[p]
        ).wait()
        @pl.when(b + 1 < NBLK)
        def _():
            pltpu.async_copy(rowp.at[pl.ds(ebase + (b + 1) * BLK, BLK)],
                             rowb.at[1 - p], isem.at[1 - p])
        @pl.when(b > 0)
        def _():
            for j in range(NB):
                pltpu.make_async_copy(
                    onesv, dacc.at[idxb.at[j]], ssem.at[j]).wait()
        for j in range(NB):
            for i in range(K // 16):
                r = rowb[p, pl.ds(j * K + i * 16, 16)]
                lo = r - c_lo
                ok = (lo >= 0) & (lo < H)
                idxb[j, pl.ds(i * 16, 16)] = jnp.where(ok, lo, H)
        for j in range(NB):
            pltpu.async_copy(onesv, dacc.at[idxb.at[j]], ssem.at[j], add=True)
        return carry

    lax.fori_loop(0, NBLK, block, 0)
    for j in range(NB):
        pltpu.make_async_copy(onesv, dacc.at[idxb.at[j]], ssem.at[j]).wait()
    plsc.subcore_barrier()
    ob = jnp.minimum(s * CH, H - CH)
    pltpu.sync_copy(dacc.at[pl.ds(ob, CH)], deg16.at[pl.ds(c_lo + ob, CH)])


@functools.partial(
    pl.kernel,
    out_type=_f32((NN, DD)),
    mesh=_mesh,
    compiler_params=pltpu.CompilerParams(use_tc_tiling_on_sc=False),
    scratch_types=[
        pltpu.VMEM_SHARED((HP, DD), jnp.float32),
        pltpu.VMEM((2, BLK), jnp.int32),
        pltpu.VMEM((2, BLK), jnp.int32),
        pltpu.VMEM((NB, K), jnp.int32),
        pltpu.VMEM((NB, K, DD), jnp.float32),
        pltpu.SemaphoreType.DMA((2,)),
        pltpu.SemaphoreType.DMA((NB,)),
        pltpu.SemaphoreType.DMA((NB,)),
    ],
)
def _spmm_kernel(y, colp, rowp, zrs, out,
                 acc, colb, rowb, idxb, gbuf, isem, gsem, ssem):
    c = lax.axis_index("c")
    s = lax.axis_index("s")
    c_lo = c * H
    ebase = s * PER_SUB
    pltpu.sync_copy(zrs.at[pl.ds(s * CH, CH)], acc.at[pl.ds(s * CH, CH)])
    plsc.subcore_barrier()

    pltpu.async_copy(colp.at[pl.ds(ebase, BLK)], colb.at[0], isem.at[0])
    pltpu.async_copy(rowp.at[pl.ds(ebase, BLK)], rowb.at[0], isem.at[0])

    def block(b, carry):
        p = jnp.bitwise_and(b, 1)
        base = ebase + b * BLK
        pltpu.make_async_copy(
            colp.at[pl.ds(base, BLK)], colb.at[p], isem.at[p]).wait()
        pltpu.make_async_copy(
            rowp.at[pl.ds(base, BLK)], rowb.at[p], isem.at[p]).wait()
        @pl.when(b + 1 < NBLK)
        def _():
            nbase = base + BLK
            pltpu.async_copy(colp.at[pl.ds(nbase, BLK)],
                             colb.at[1 - p], isem.at[1 - p])
            pltpu.async_copy(rowp.at[pl.ds(nbase, BLK)],
                             rowb.at[1 - p], isem.at[1 - p])
        @pl.when(b > 0)
        def _():
            for j in range(NB):
                pltpu.make_async_copy(
                    gbuf.at[j], acc.at[idxb.at[j]], ssem.at[j]).wait()
        for j in range(NB):
            for i in range(K // 16):
                r = rowb[p, pl.ds(j * K + i * 16, 16)]
                lo = r - c_lo
                ok = (lo >= 0) & (lo < H)
                idxb[j, pl.ds(i * 16, 16)] = jnp.where(ok, lo, H)
        gd = [pltpu.async_copy(y.at[colb.at[p, pl.ds(j * K, K)]],
                               gbuf.at[j], gsem.at[j]) for j in range(NB)]
        for j in range(NB):
            gd[j].wait()
            pltpu.async_copy(gbuf.at[j], acc.at[idxb.at[j]],
                             ssem.at[j], add=True)
        return carry

    lax.fori_loop(0, NBLK, block, 0)
    for j in range(NB):
        pltpu.make_async_copy(
            gbuf.at[j], acc.at[idxb.at[j]], ssem.at[j]).wait()
    plsc.subcore_barrier()
    ob = jnp.minimum(s * CH, H - CH)
    pltpu.sync_copy(acc.at[pl.ds(ob, CH)], out.at[pl.ds(c_lo + ob, CH)])


@functools.partial(
    pl.kernel,
    out_type=(_f32((BB, DD)), _f32((BB, DD)), _f32((BB, DD))),
    mesh=_mesh,
    compiler_params=pltpu.CompilerParams(use_tc_tiling_on_sc=False),
    scratch_types=[
        pltpu.VMEM((128,), jnp.int32),
        pltpu.VMEM((128, DD), jnp.float32),
        pltpu.SemaphoreType.DMA,
    ],
)
def _bpr_gather_kernel(S, users, items, negs, anc, pos, neg, idxv, buf, sem):
    c = lax.axis_index("c")
    s = lax.axis_index("s")
    base = (s * 2 + c) * 128

    pltpu.sync_copy(users.at[pl.ds(base, 128)], idxv)
    pltpu.async_copy(S.at[idxv], buf, sem).wait()
    pltpu.sync_copy(buf, anc.at[pl.ds(base, 128)])

    for src, dst in ((items, pos), (negs, neg)):
        pltpu.sync_copy(src.at[pl.ds(base, 128)], idxv)
        for j in range(8):
            idxv[pl.ds(j * 16, 16)] = idxv[pl.ds(j * 16, 16)] + N_U
        pltpu.async_copy(S.at[idxv], buf, sem).wait()
        pltpu.sync_copy(buf, dst.at[pl.ds(base, 128)])


_RB = 2000


def _prep_body(deg_ref, x0_ref, dis_ref, y0_ref):
    d = jnp.clip(lax.rsqrt(deg_ref[:, 0:1] + 1e-6), 0.0, 10.0)
    dis_ref[...] = d
    y0_ref[...] = d * x0_ref[...]


_prep = pl.pallas_call(
    _prep_body,
    grid=(NN // _RB,),
    in_specs=[
        pl.BlockSpec((_RB, 16), lambda i: (i, 0)),
        pl.BlockSpec((_RB, DD), lambda i: (i, 0)),
    ],
    out_specs=[
        pl.BlockSpec((_RB, 1), lambda i: (i, 0)),
        pl.BlockSpec((_RB, DD), lambda i: (i, 0)),
    ],
    out_shape=[_f32((NN, 1)), _f32((NN, DD))],
)


def _scale_body(acc_ref, dis_ref, s_ref, snew_ref, y_ref):
    d = dis_ref[...]
    da = d * acc_ref[...]
    snew_ref[...] = s_ref[...] + da
    y_ref[...] = d * da


_scale = pl.pallas_call(
    _scale_body,
    grid=(NN // _RB,),
    in_specs=[
        pl.BlockSpec((_RB, DD), lambda i: (i, 0)),
        pl.BlockSpec((_RB, 1), lambda i: (i, 0)),
        pl.BlockSpec((_RB, DD), lambda i: (i, 0)),
    ],
    out_specs=[
        pl.BlockSpec((_RB, DD), lambda i: (i, 0)),
        pl.BlockSpec((_RB, DD), lambda i: (i, 0)),
    ],
    out_shape=[_f32((NN, DD)), _f32((NN, DD))],
)


def _loss_body(a_ref, p_ref, n_ref, o_ref):
    a = a_ref[...]
    p = p_ref[...]
    n = n_ref[...]
    diff = jnp.sum(a * p, axis=-1) - jnp.sum(a * n, axis=-1)
    bpr = -jnp.sum(jnp.log(jax.nn.sigmoid(diff) + 1e-12)) / float(BB)
    reg = 0.5 * (jnp.sum(a * a) + jnp.sum(p * p) + jnp.sum(n * n)) / float(BB)
    o_ref[...] = (bpr + reg).reshape(1, 1)


_loss = pl.pallas_call(
    _loss_body,
    out_shape=_f32((1, 1)),
)


def kernel(user_emb, item_emb, edge_index, users, items, neg_items):
    x0 = jnp.concatenate([user_emb, item_emb], axis=0)
    pad = EP - EE
    colp = jnp.concatenate([edge_index[1], jnp.zeros((pad,), jnp.int32)])
    rowp = jnp.concatenate([edge_index[0], jnp.full((pad,), -1, jnp.int32)])
    zrs = jnp.zeros((HP, DD), jnp.float32)
    zrs16 = jnp.zeros((HP, 16), jnp.float32)

    deg16 = _deg_kernel(rowp, zrs16)
    dis, y = _prep(deg16, x0)
    s_sum = x0
    for _ in range(3):
        acc = _spmm_kernel(y, colp, rowp, zrs)
        s_sum, y = _scale(acc, dis, s_sum)
    anc, pos, neg = _bpr_gather_kernel(s_sum, users, items, neg_items)
    return _loss(anc, pos, neg)[0, 0]
